# baseline (speedup 1.0000x reference)
import jax
import jax.numpy as jnp
from jax.experimental import pallas as pl
from jax.experimental.pallas import tpu as pltpu

_D = 64
_K = 1024
_NCB = 4
_ROWS = 4096
_CHUNKS = 2


def _split3(cb):
    b1 = cb.astype(jnp.bfloat16)
    r1 = cb - b1.astype(jnp.float32)
    b2 = r1.astype(jnp.bfloat16)
    r2 = r1 - b2.astype(jnp.float32)
    b3 = r2.astype(jnp.bfloat16)
    return b1, b2, b3


def _rvq_body(x_ref, win_ref, bin_ref, wout_ref, bout_ref, csq_ref,
              ct0_ref, ct1_ref, ct2_ref, ct3_ref,
              *rest):
    g_refs = rest[:4]
    recon_ref, idx_ref = rest[4], rest[5]
    x = x_ref[...]
    z = jnp.dot(x.astype(jnp.bfloat16), win_ref[...],
                preferred_element_type=jnp.float32) + bin_ref[...]
    cts = (ct0_ref, ct1_ref, ct2_ref, ct3_ref)
    lanes = jax.lax.broadcasted_iota(jnp.int32, (_ROWS, _K), 1)
    residual = z
    z_q = jnp.zeros_like(z)
    for k in range(_NCB):
        rsq = jnp.sum(residual * residual, axis=1, keepdims=True)
        key = (rsq + jnp.dot(residual.astype(jnp.bfloat16), cts[k][...],
                             preferred_element_type=jnp.float32)
               ) + csq_ref[k:k + 1, :]
        m = jnp.min(key, axis=1, keepdims=True)
        ind = jnp.min(jnp.where(key == m, lanes, _K), axis=1,
                      keepdims=True)
        onehot = (lanes == ind).astype(jnp.bfloat16)
        qcat = jnp.dot(onehot, g_refs[k][...],
                       preferred_element_type=jnp.float32)
        q = (qcat[:, :_D] + qcat[:, _D:2 * _D]) + qcat[:, 2 * _D:3 * _D]
        residual = residual - q
        z_q = z_q + q
        idx_ref[:, k:k + 1] = ind
    recon_ref[...] = (
        jnp.dot(z_q.astype(jnp.bfloat16), wout_ref[...],
                preferred_element_type=jnp.float32)
        + bout_ref[...])


def kernel(mel_frame, W_in, b_in, W_out, b_out, cb0, cb1, cb2, cb3):
    Bb, Tt, Mm = mel_frame.shape
    N = Bb * Tt
    x = mel_frame.reshape(N, Mm)

    def full(shape):
        return pl.BlockSpec(shape, lambda i: (0, 0))

    splits = [jnp.concatenate(_split3(cb), axis=1)
              for cb in (cb0, cb1, cb2, cb3)]
    csq = jnp.stack([jnp.sum(cb * cb, axis=-1)
                     for cb in (cb0, cb1, cb2, cb3)])

    recon, inds = pl.pallas_call(
        _rvq_body,
        grid=(N // _ROWS,),
        in_specs=[
            pl.BlockSpec((_ROWS, Mm), lambda i: (i, 0)),
            full((Mm, _D)), full((1, _D)), full((_D, Mm)), full((1, Mm)),
            full((_NCB, _K)),
            full((_D, _K)), full((_D, _K)), full((_D, _K)), full((_D, _K)),
        ] + [full((_K, 3 * _D))] * 4,
        out_specs=[
            pl.BlockSpec((_ROWS, Mm), lambda i: (i, 0)),
            pl.BlockSpec((_ROWS, _NCB), lambda i: (i, 0)),
        ],
        out_shape=[
            jax.ShapeDtypeStruct((N, Mm), jnp.float32),
            jax.ShapeDtypeStruct((N, _NCB), jnp.int32),
        ],
        compiler_params=pltpu.CompilerParams(
            dimension_semantics=("parallel",)),
    )(x, W_in.astype(jnp.bfloat16), b_in.reshape(1, _D),
      W_out.astype(jnp.bfloat16), b_out.reshape(1, Mm), csq,
      (-2.0 * cb0.T).astype(jnp.bfloat16), (-2.0 * cb1.T).astype(jnp.bfloat16),
      (-2.0 * cb2.T).astype(jnp.bfloat16), (-2.0 * cb3.T).astype(jnp.bfloat16),
      *splits)
    return recon.reshape(Bb, Tt, Mm), inds.reshape(Bb, Tt, _NCB)


# speedup vs baseline: 1.2014x; 1.2014x over previous
import jax
import jax.numpy as jnp
from jax.experimental import pallas as pl
from jax.experimental.pallas import tpu as pltpu

_D = 64
_K = 1024
_NCB = 4
_ROWS = 2048


def _split3(cb):
    b1 = cb.astype(jnp.bfloat16)
    r1 = cb - b1.astype(jnp.float32)
    b2 = r1.astype(jnp.bfloat16)
    r2 = r1 - b2.astype(jnp.float32)
    b3 = r2.astype(jnp.bfloat16)
    return b1, b2, b3


def _rvq_body(x_ref, win_ref, bin_ref, wout_ref, bout_ref, csq_ref,
              ct0_ref, ct1_ref, ct2_ref, ct3_ref,
              *rest):
    g_refs = rest[:4]
    recon_ref, idx_ref = rest[4], rest[5]
    x = x_ref[...]
    z = jnp.dot(x.astype(jnp.bfloat16), win_ref[...],
                preferred_element_type=jnp.float32) + bin_ref[...]
    cts = (ct0_ref, ct1_ref, ct2_ref, ct3_ref)
    lanes = jax.lax.broadcasted_iota(jnp.int32, (_ROWS, _K), 1)
    residual = z
    z_q = jnp.zeros_like(z)
    for k in range(_NCB):
        rsq = jnp.sum(residual * residual, axis=1, keepdims=True)
        key = (rsq + jnp.dot(residual.astype(jnp.bfloat16), cts[k][...],
                             preferred_element_type=jnp.float32)
               ) + csq_ref[k:k + 1, :]
        m = jnp.min(key, axis=1, keepdims=True)
        ind = jnp.min(jnp.where(key == m, lanes, _K), axis=1,
                      keepdims=True)
        onehot = (lanes == ind).astype(jnp.bfloat16)
        qcat = jnp.dot(onehot, g_refs[k][...],
                       preferred_element_type=jnp.float32)
        q = (qcat[:, :_D] + qcat[:, _D:2 * _D]) + qcat[:, 2 * _D:3 * _D]
        residual = residual - q
        z_q = z_q + q
        idx_ref[:, k:k + 1] = ind
    recon_ref[...] = (
        jnp.dot(z_q.astype(jnp.bfloat16), wout_ref[...],
                preferred_element_type=jnp.float32)
        + bout_ref[...])


def kernel(mel_frame, W_in, b_in, W_out, b_out, cb0, cb1, cb2, cb3):
    Bb, Tt, Mm = mel_frame.shape
    N = Bb * Tt
    x = mel_frame.reshape(N, Mm)

    def full(shape):
        return pl.BlockSpec(shape, lambda i: (0, 0))

    splits = [jnp.concatenate(_split3(cb), axis=1)
              for cb in (cb0, cb1, cb2, cb3)]
    csq = jnp.stack([jnp.sum(cb * cb, axis=-1)
                     for cb in (cb0, cb1, cb2, cb3)])

    recon, inds = pl.pallas_call(
        _rvq_body,
        grid=(N // _ROWS,),
        in_specs=[
            pl.BlockSpec((_ROWS, Mm), lambda i: (i, 0)),
            full((Mm, _D)), full((1, _D)), full((_D, Mm)), full((1, Mm)),
            full((_NCB, _K)),
            full((_D, _K)), full((_D, _K)), full((_D, _K)), full((_D, _K)),
        ] + [full((_K, 3 * _D))] * 4,
        out_specs=[
            pl.BlockSpec((_ROWS, Mm), lambda i: (i, 0)),
            pl.BlockSpec((_ROWS, _NCB), lambda i: (i, 0)),
        ],
        out_shape=[
            jax.ShapeDtypeStruct((N, Mm), jnp.float32),
            jax.ShapeDtypeStruct((N, _NCB), jnp.int32),
        ],
        compiler_params=pltpu.CompilerParams(
            dimension_semantics=("parallel",)),
    )(x, W_in.astype(jnp.bfloat16), b_in.reshape(1, _D),
      W_out.astype(jnp.bfloat16), b_out.reshape(1, Mm), csq,
      (-2.0 * cb0.T).astype(jnp.bfloat16), (-2.0 * cb1.T).astype(jnp.bfloat16),
      (-2.0 * cb2.T).astype(jnp.bfloat16), (-2.0 * cb3.T).astype(jnp.bfloat16),
      *splits)
    return recon.reshape(Bb, Tt, Mm), inds.reshape(Bb, Tt, _NCB)
